# baseline (speedup 1.0000x reference)
import functools

import jax
import jax.numpy as jnp
from jax import lax
from jax.experimental import pallas as pl
from jax.experimental.pallas import tpu as pltpu
from jax.experimental.pallas import tpu_sc as plsc

B = 16
N = 100000
D = 32
J = 200
CPG = 7
NWORK = 32
RING = 24

_mesh = plsc.VectorSubcoreMesh(core_axis_name="c", subcore_axis_name="s")


@functools.partial(
    pl.kernel,
    mesh=_mesh,
    compiler_params=pltpu.CompilerParams(
        needs_layout_passes=False, use_tc_tiling_on_sc=True),
    out_type=jax.ShapeDtypeStruct((NWORK, D, 128), jnp.float32),
    scratch_types=[
        pltpu.VMEM((B * J,), jnp.int32),
        pltpu.VMEM((RING, D, 128), jnp.float32),
        pltpu.VMEM((D, 128), jnp.float32),
        pltpu.SemaphoreType.DMA,
        pltpu.SemaphoreType.DMA,
    ],
)
def _select_kernel(x_hbm, idx_hbm, out_hbm, idx_v, slab_v, out_v, gsem, osem):
    wid = lax.axis_index("s") * 2 + lax.axis_index("c")

    pltpu.sync_copy(idx_hbm, idx_v)
    lanes = lax.iota(jnp.int32, 16)
    j0 = wid * CPG
    ns = []
    for c in range(CPG):
        j = jnp.minimum(j0 + c, J - 1)
        col = plsc.load_gather(idx_v, [lanes * J + j])
        srt = lax.sort(col)
        for b in range(B):
            ns.append(srt[b])

    def issue(g):
        n = ns[g]
        t = pl.multiple_of(n & -128, 128)
        b = g % B
        return pltpu.async_copy(
            x_hbm.at[b, :, pl.ds(t, 128)], slab_v.at[g % RING], gsem)

    def extract(g, cp):
        cp.wait()
        n = ns[g]
        off = jnp.broadcast_to(n & 127, (16,)).astype(jnp.int32)
        gv = jnp.full((16,), g % RING, jnp.int32)
        v0 = plsc.load_gather(slab_v, [gv, lanes, off])
        v1 = plsc.load_gather(slab_v, [gv, lanes + 16, off])
        c, b = g // B, g % B
        base = c * 512 + b * 32
        r, q = base // 128, base % 128
        out_v[r, pl.ds(q, 16)] = v0
        out_v[r, pl.ds(q + 16, 16)] = v1

    total = CPG * B
    pending = [issue(g) for g in range(RING)]
    for g in range(total):
        extract(g, pending[g % RING])
        if g + RING < total:
            pending[(g + RING) % RING] = issue(g + RING)
    pltpu.async_copy(out_v, out_hbm.at[wid], osem).wait()


@jax.jit
def kernel(X, idx):
    Xt = X.transpose(0, 2, 1)
    idxf = idx.astype(jnp.int32).reshape(-1)
    out32 = _select_kernel(Xt, idxf)
    out = out32.reshape(NWORK, 8, B, D)[:, :CPG]
    out = out.transpose(2, 0, 1, 3).reshape(B, NWORK * CPG, D)
    return out[:, :J, :]


# speedup vs baseline: 1.0430x; 1.0430x over previous
import functools

import jax
import jax.numpy as jnp
from jax import lax
from jax.experimental import pallas as pl
from jax.experimental.pallas import tpu as pltpu
from jax.experimental.pallas import tpu_sc as plsc

B = 16
N = 100000
D = 32
J = 200
NWORK = 32
NBIG = 8
RING = 16

_mesh = plsc.VectorSubcoreMesh(core_axis_name="c", subcore_axis_name="s")


@functools.partial(
    pl.kernel,
    mesh=_mesh,
    compiler_params=pltpu.CompilerParams(
        needs_layout_passes=False, use_tc_tiling_on_sc=True),
    out_type=jax.ShapeDtypeStruct((NWORK, D, 128), jnp.float32),
    scratch_types=[
        pltpu.VMEM((B * J,), jnp.int32),
        pltpu.VMEM((RING, D, 128), jnp.float32),
        pltpu.VMEM((D, 128), jnp.float32),
        pltpu.SemaphoreType.DMA,
        pltpu.SemaphoreType.DMA,
    ],
)
def _select_kernel(x_hbm, idx_hbm, out_hbm, idx_v, slab_v, out_v, gsem, osem):
    wid = lax.axis_index("s") * 2 + lax.axis_index("c")

    pltpu.sync_copy(idx_hbm, idx_v)
    lanes = lax.iota(jnp.int32, 16)
    is_big = wid < NBIG
    j0 = jnp.where(is_big, wid * 7, NBIG * 7 + (wid - NBIG) * 6)
    ncols = jnp.where(is_big, 7, 6)
    ns = []
    for c in range(7):
        j = jnp.minimum(j0 + c, j0 + ncols - 1)
        col = plsc.load_gather(idx_v, [lanes * J + j])
        srt = lax.sort(col)
        for b in range(B):
            ns.append(srt[b])

    def issue(g):
        n = ns[g]
        t = pl.multiple_of(n & -128, 128)
        b = g % B
        return pltpu.async_copy(
            x_hbm.at[b, :, pl.ds(t, 128)], slab_v.at[g % RING], gsem)

    def extract(g, cp):
        cp.wait()
        n = ns[g]
        off = jnp.broadcast_to(n & 127, (16,)).astype(jnp.int32)
        gv = jnp.full((16,), g % RING, jnp.int32)
        v0 = plsc.load_gather(slab_v, [gv, lanes, off])
        v1 = plsc.load_gather(slab_v, [gv, lanes + 16, off])
        c, b = g // B, g % B
        base = c * 512 + b * 32
        r, q = base // 128, base % 128
        out_v[r, pl.ds(q, 16)] = v0
        out_v[r, pl.ds(q + 16, 16)] = v1

    total = 7 * B
    pending = [issue(g) for g in range(RING)]
    for g in range(total):
        extract(g, pending[g % RING])
        if g + RING < total:
            pending[(g + RING) % RING] = issue(g + RING)
    pltpu.async_copy(out_v, out_hbm.at[wid], osem).wait()


@jax.jit
def kernel(X, idx):
    Xt = X.transpose(0, 2, 1)
    idxf = idx.astype(jnp.int32).reshape(-1)
    out32 = _select_kernel(Xt, idxf)
    cols = out32.reshape(NWORK, 8, B, D)
    big = cols[:NBIG, :7].reshape(NBIG * 7, B, D)
    small = cols[NBIG:, :6].reshape((NWORK - NBIG) * 6, B, D)
    out = jnp.concatenate([big, small], axis=0)
    return out.transpose(1, 0, 2)


# speedup vs baseline: 1.0631x; 1.0193x over previous
import functools

import jax
import jax.numpy as jnp
from jax import lax
from jax.experimental import pallas as pl
from jax.experimental.pallas import tpu as pltpu
from jax.experimental.pallas import tpu_sc as plsc

B = 16
N = 100000
D = 32
J = 200
CPG = 8
NWORK = J // CPG
RING = 24

_mesh = plsc.VectorSubcoreMesh(core_axis_name="c", subcore_axis_name="s")


@functools.partial(
    pl.kernel,
    mesh=_mesh,
    compiler_params=pltpu.CompilerParams(
        needs_layout_passes=False, use_tc_tiling_on_sc=True),
    out_type=jax.ShapeDtypeStruct((NWORK, D, 128), jnp.float32),
    scratch_types=[
        pltpu.VMEM((B * J,), jnp.int32),
        pltpu.VMEM((RING, D, 128), jnp.float32),
        pltpu.VMEM((D, 128), jnp.float32),
        pltpu.SemaphoreType.DMA,
        pltpu.SemaphoreType.DMA,
    ],
)
def _select_kernel(x_hbm, idx_hbm, out_hbm, idx_v, slab_v, out_v, gsem, osem):
    wid = lax.axis_index("s") * 2 + lax.axis_index("c")

    @pl.when(wid < NWORK)
    def _():
        pltpu.sync_copy(idx_hbm, idx_v)
        lanes = lax.iota(jnp.int32, 16)
        j0 = wid * CPG
        ns = []
        for c in range(CPG):
            col = plsc.load_gather(idx_v, [lanes * J + j0 + c])
            srt = lax.sort(col)
            for b in range(B):
                ns.append(srt[b])

        def issue(g):
            n = ns[g]
            t = pl.multiple_of(n & -128, 128)
            b = g % B
            return pltpu.async_copy(
                x_hbm.at[b, :, pl.ds(t, 128)], slab_v.at[g % RING], gsem)

        def extract(g, cp):
            cp.wait()
            n = ns[g]
            off = jnp.broadcast_to(n & 127, (16,)).astype(jnp.int32)
            gv = jnp.full((16,), g % RING, jnp.int32)
            v0 = plsc.load_gather(slab_v, [gv, lanes, off])
            v1 = plsc.load_gather(slab_v, [gv, lanes + 16, off])
            c, b = g // B, g % B
            base = c * 512 + b * 32
            r, q = base // 128, base % 128
            out_v[r, pl.ds(q, 16)] = v0
            out_v[r, pl.ds(q + 16, 16)] = v1

        total = CPG * B
        pending = [issue(g) for g in range(RING)]
        for g in range(total):
            extract(g, pending[g % RING])
            if g + RING < total:
                pending[(g + RING) % RING] = issue(g + RING)
        pltpu.async_copy(out_v, out_hbm.at[wid], osem).wait()


@jax.jit
def kernel(X, idx):
    Xt = X.transpose(0, 2, 1)
    idxf = idx.astype(jnp.int32).reshape(-1)
    out25 = _select_kernel(Xt, idxf)
    out = out25.reshape(NWORK, CPG, B, D).transpose(2, 0, 1, 3)
    return out.reshape(B, J, D)


# speedup vs baseline: 1.0746x; 1.0107x over previous
import functools

import jax
import jax.numpy as jnp
from jax import lax
from jax.experimental import pallas as pl
from jax.experimental.pallas import tpu as pltpu
from jax.experimental.pallas import tpu_sc as plsc

B = 16
N = 100000
D = 32
J = 200
CPG = 8
NWORK = J // CPG
RING = 16

_mesh = plsc.VectorSubcoreMesh(core_axis_name="c", subcore_axis_name="s")


@functools.partial(
    pl.kernel,
    mesh=_mesh,
    compiler_params=pltpu.CompilerParams(
        needs_layout_passes=False, use_tc_tiling_on_sc=True),
    out_type=jax.ShapeDtypeStruct((NWORK, D, 128), jnp.float32),
    scratch_types=[
        pltpu.VMEM((B * J,), jnp.int32),
        pltpu.VMEM((RING, D, 128), jnp.float32),
        pltpu.VMEM((D, 128), jnp.float32),
        pltpu.SemaphoreType.DMA,
        pltpu.SemaphoreType.DMA,
    ],
)
def _select_kernel(x_hbm, idx_hbm, out_hbm, idx_v, slab_v, out_v, gsem, osem):
    wid = lax.axis_index("s") * 2 + lax.axis_index("c")

    @pl.when(wid < NWORK)
    def _():
        pltpu.sync_copy(idx_hbm, idx_v)
        lanes = lax.iota(jnp.int32, 16)
        j0 = wid * CPG
        ns = []
        for c in range(CPG):
            col = plsc.load_gather(idx_v, [lanes * J + j0 + c])
            srt = lax.sort(col)
            for b in range(B):
                ns.append(srt[b])

        def issue(g):
            n = ns[g]
            t = pl.multiple_of(n & -128, 128)
            b = g % B
            return pltpu.async_copy(
                x_hbm.at[b, :, pl.ds(t, 128)], slab_v.at[g % RING], gsem)

        def extract(g, cp):
            cp.wait()
            n = ns[g]
            off = jnp.broadcast_to(n & 127, (16,)).astype(jnp.int32)
            gv = jnp.full((16,), g % RING, jnp.int32)
            v0 = plsc.load_gather(slab_v, [gv, lanes, off])
            v1 = plsc.load_gather(slab_v, [gv, lanes + 16, off])
            c, b = g // B, g % B
            base = c * 512 + b * 32
            r, q = base // 128, base % 128
            out_v[r, pl.ds(q, 16)] = v0
            out_v[r, pl.ds(q + 16, 16)] = v1

        total = CPG * B
        pending = [issue(g) for g in range(RING)]
        for g in range(total):
            extract(g, pending[g % RING])
            if g + RING < total:
                pending[(g + RING) % RING] = issue(g + RING)
        pltpu.async_copy(out_v, out_hbm.at[wid], osem).wait()


@jax.jit
def kernel(X, idx):
    Xt = X.transpose(0, 2, 1)
    idxf = idx.astype(jnp.int32).reshape(-1)
    out25 = _select_kernel(Xt, idxf)
    out = out25.reshape(NWORK, CPG, B, D).transpose(2, 0, 1, 3)
    return out.reshape(B, J, D)
